# baseline (speedup 1.0000x reference)
import functools
import math

import jax
import jax.numpy as jnp
from jax import lax
from jax.experimental import pallas as pl
from jax.experimental.pallas import tpu as pltpu
from jax.experimental.pallas import tpu_sc as plsc

NC = 2
NS = 16
LK = 128

_BN_C = (1.0 + 1e-5) ** -0.5
_SC_PARAMS = pltpu.CompilerParams(use_tc_tiling_on_sc=False)


def _seg_rows(nrows, w, ch):
    mesh = plsc.VectorSubcoreMesh(core_axis_name="c", subcore_axis_name="s",
                                  num_cores=NC, num_subcores=NS)
    rpt = nrows // NS

    nb = 4
    lag = 2

    @functools.partial(
        pl.kernel,
        out_type=jax.ShapeDtypeStruct((NC, nrows, w), jnp.float32),
        mesh=mesh,
        compiler_params=_SC_PARAMS,
        scratch_types=[
            pltpu.VMEM((ch, LK), jnp.int32),
            pltpu.VMEM((ch, LK), jnp.int32),
            [pltpu.VMEM((LK, w), jnp.float32)] * nb,
            pltpu.VMEM_SHARED((nrows, w), jnp.float32),
            [pltpu.SemaphoreType.DMA] * nb,
            [pltpu.SemaphoreType.DMA] * nb,
        ],
    )
    def k(table_h, src_h, dst_h, z_h, out_h, src_v, dst_v, rows, acc,
          gsem, ssem):
        c = lax.axis_index("c")
        s = lax.axis_index("s")
        pltpu.sync_copy(src_h.at[c, s], src_v)
        pltpu.sync_copy(dst_h.at[c, s], dst_v)
        for p in range(lag):
            pltpu.async_copy(table_h.at[src_v.at[p]], rows[p], gsem[p])
        pltpu.sync_copy(z_h, acc.at[pl.ds(s * rpt, rpt)])
        plsc.subcore_barrier()

        @pl.loop(0, ch, step=nb)
        def _(j):
            for b in range(nb):
                cidx = j + b
                pltpu.make_async_copy(table_h.at[src_v.at[cidx]], rows[b],
                                      gsem[b]).wait()
                pltpu.async_copy(rows[b], acc.at[dst_v.at[cidx]], ssem[b],
                                 add=True)
                fc = cidx + lag
                fb = (b + lag) % nb

                @pl.when(fc < ch)
                def _():
                    @pl.when(fc >= nb)
                    def _():
                        pltpu.make_async_copy(
                            rows[fb], acc.at[dst_v.at[cidx]], ssem[fb]).wait()
                    pltpu.async_copy(table_h.at[src_v.at[fc]], rows[fb],
                                     gsem[fb])

        for b in range(nb):
            pltpu.make_async_copy(rows[b], acc.at[dst_v.at[0]],
                                  ssem[b]).wait()
        plsc.subcore_barrier()
        pltpu.sync_copy(acc.at[pl.ds(s * rpt, rpt)],
                        out_h.at[c, pl.ds(s * rpt, rpt)])

    return k


def _deg_rows(nrows, w, ch):
    mesh = plsc.VectorSubcoreMesh(core_axis_name="c", subcore_axis_name="s",
                                  num_cores=NC, num_subcores=NS)
    rpt = nrows // NS

    @functools.partial(
        pl.kernel,
        out_type=jax.ShapeDtypeStruct((NC, nrows, w), jnp.float32),
        mesh=mesh,
        compiler_params=_SC_PARAMS,
        scratch_types=[
            pltpu.VMEM((ch, LK), jnp.int32),
            pltpu.VMEM((LK, w), jnp.float32),
            pltpu.VMEM_SHARED((nrows, w), jnp.float32),
        ],
    )
    def k(dst_h, ones_h, z_h, out_h, dst_v, ones_v, acc):
        c = lax.axis_index("c")
        s = lax.axis_index("s")
        pltpu.sync_copy(dst_h.at[c, s], dst_v)
        pltpu.sync_copy(ones_h, ones_v)
        pltpu.sync_copy(z_h, acc.at[pl.ds(s * rpt, rpt)])
        plsc.subcore_barrier()

        @pl.loop(0, ch)
        def _(j):
            pltpu.sync_copy(ones_v, acc.at[dst_v.at[j]], add=True)

        plsc.subcore_barrier()
        pltpu.sync_copy(acc.at[pl.ds(s * rpt, rpt)],
                        out_h.at[c, pl.ds(s * rpt, rpt)])

    return k


def _enc_body(x_ref, w_ref, b_ref, h_ref):
    h_ref[...] = jnp.maximum(
        jnp.dot(x_ref[...], w_ref[...], preferred_element_type=jnp.float32)
        + b_ref[...], 0.0)


def _pre0_body(deg_ref, h_ref, w_ref, dis_ref, hw_ref, *, n):
    deg = deg_ref[0, :n, 0:1] + deg_ref[1, :n, 0:1] + 1.0
    dis = lax.rsqrt(deg)
    dis_ref[...] = dis
    hw_ref[...] = jnp.dot(h_ref[...], w_ref[...],
                          preferred_element_type=jnp.float32) * dis


def _mid_body(sp_ref, hwp_ref, dis_ref, hres_ref, bc_ref, g_ref, be_ref,
              wn_ref, h_ref, hwn_ref, *, n, residual):
    dis = dis_ref[...]
    ssum = sp_ref[0, :n, :] + sp_ref[1, :n, :] + hwp_ref[...]
    agg = ssum * dis + bc_ref[...]
    h = jnp.maximum(g_ref[...] * (agg * _BN_C) + be_ref[...], 0.0)
    if residual:
        h = h + hres_ref[...]
    h_ref[...] = h
    hwn_ref[...] = jnp.dot(h, wn_ref[...],
                           preferred_element_type=jnp.float32) * dis


def _head_body(sp_ref, hwp_ref, dis_ref, hres_ref, bc_ref, g_ref, be_ref,
               wa1_ref, ba1_ref, wa2_ref, ba2_ref, wk1_ref, bk1_ref,
               wk2_ref, bk2_ref, out_ref, *, n):
    dis = dis_ref[...]
    ssum = sp_ref[0, :n, :] + sp_ref[1, :n, :] + hwp_ref[...]
    agg = ssum * dis + bc_ref[...]
    h = jnp.maximum(g_ref[...] * (agg * _BN_C) + be_ref[...], 0.0)
    h = h + hres_ref[...]
    a = jnp.dot(jnp.tanh(jnp.dot(h, wa1_ref[...],
                                 preferred_element_type=jnp.float32)
                         + ba1_ref[...]),
                wa2_ref[...], preferred_element_type=jnp.float32) + ba2_ref[...]
    a = a - jnp.max(a, axis=0, keepdims=True)
    ea = jnp.exp(a)
    a = ea / jnp.sum(ea, axis=0, keepdims=True)
    h = h * a
    out_ref[...] = jnp.dot(
        jnp.maximum(jnp.dot(h, wk1_ref[...],
                            preferred_element_type=jnp.float32)
                    + bk1_ref[...], 0.0),
        wk2_ref[...], preferred_element_type=jnp.float32) + bk2_ref[...]


def _tc(body, out_shape, **kw):
    return pl.pallas_call(functools.partial(body, **kw), out_shape=out_shape)


def kernel(x, edge_index, W_ne, b_ne, Wc0, bc0, g0, be0, Wc1, bc1, g1, be1,
           Wc2, bc2, g2, be2, Wa1, ba1, Wa2, ba2, Wk1, bk1, Wk2, bk2):
    n = x.shape[0]
    h_dim = W_ne.shape[1]
    e = edge_index.shape[1]

    ch = 4 * math.ceil(e / (NC * NS * LK * 4))
    epad = NC * NS * ch * LK
    rpt = 8 * math.ceil((n + 8) / (NS * 8))
    nrows = NS * rpt

    src = edge_index[0]
    dst = edge_index[1]
    src_r = jnp.pad(src, (0, epad - e)).reshape(NC, NS, ch, LK)
    dst_r = jnp.pad(dst, (0, epad - e), constant_values=n).reshape(
        NC, NS, ch, LK)

    zrows = jnp.zeros((rpt, h_dim), jnp.float32)
    zdeg = jnp.zeros((rpt, 16), jnp.float32)
    ones16 = jnp.ones((LK, 16), jnp.float32)

    seg = _seg_rows(nrows, h_dim, ch)
    deg_parts = _deg_rows(nrows, 16, ch)(dst_r, ones16, zdeg)

    h0 = _tc(_enc_body, jax.ShapeDtypeStruct((n, h_dim), jnp.float32))(
        x, W_ne, b_ne)

    dis, hw0 = _tc(_pre0_body,
                   (jax.ShapeDtypeStruct((n, 1), jnp.float32),
                    jax.ShapeDtypeStruct((n, h_dim), jnp.float32)),
                   n=n)(deg_parts, h0, Wc0)

    s0 = seg(hw0, src_r, dst_r, zrows)
    h1, hw1 = _tc(_mid_body,
                  (jax.ShapeDtypeStruct((n, h_dim), jnp.float32),
                   jax.ShapeDtypeStruct((n, h_dim), jnp.float32)),
                  n=n, residual=False)(s0, hw0, dis, h0, bc0, g0, be0, Wc1)

    s1 = seg(hw1, src_r, dst_r, zrows)
    h2, hw2 = _tc(_mid_body,
                  (jax.ShapeDtypeStruct((n, h_dim), jnp.float32),
                   jax.ShapeDtypeStruct((n, h_dim), jnp.float32)),
                  n=n, residual=True)(s1, hw1, dis, h1, bc1, g1, be1, Wc2)

    s2 = seg(hw2, src_r, dst_r, zrows)
    out = _tc(_head_body, jax.ShapeDtypeStruct((n, 2), jnp.float32), n=n)(
        s2, hw2, dis, h2, bc2, g2, be2,
        Wa1, ba1, Wa2, ba2, Wk1, bk1, Wk2, bk2)
    return out


# speedup vs baseline: 1.7580x; 1.7580x over previous
import functools
import math

import jax
import jax.numpy as jnp
from jax import lax
from jax.experimental import pallas as pl
from jax.experimental.pallas import tpu as pltpu
from jax.experimental.pallas import tpu_sc as plsc

NC = 2
NS = 16
LK = 128

_BN_C = (1.0 + 1e-5) ** -0.5
_SC_PARAMS = pltpu.CompilerParams(use_tc_tiling_on_sc=False)


def _seg_rows(nrows, w, ch, n):
    mesh = plsc.VectorSubcoreMesh(core_axis_name="c", subcore_axis_name="s",
                                  num_cores=NC, num_subcores=NS)
    rpt = nrows // NS
    wh = w // 2

    nb = 4
    lag = 2

    @functools.partial(
        pl.kernel,
        out_type=jax.ShapeDtypeStruct((NC, 2, nrows, wh), jnp.float32),
        mesh=mesh,
        compiler_params=_SC_PARAMS,
        scratch_types=[
            pltpu.VMEM((ch, LK), jnp.int32),
            pltpu.VMEM((ch, LK), jnp.int32),
            [pltpu.VMEM((LK, wh), jnp.float32)] * nb,
            pltpu.VMEM_SHARED((nrows, wh), jnp.float32),
            pltpu.VMEM_SHARED((n, wh), jnp.float32),
            [pltpu.SemaphoreType.DMA] * nb,
            [pltpu.SemaphoreType.DMA] * nb,
        ],
    )
    def k(table_h, src_h, dst_h, z_h, out_h, src_v, dst_v, rows, acc, tbl,
          gsem, ssem):
        c = lax.axis_index("c")
        s = lax.axis_index("s")
        pltpu.sync_copy(src_h.at[c, s], src_v)
        pltpu.sync_copy(dst_h.at[c, s], dst_v)
        trows = 8 * math.ceil(n / (NS * 8))
        tstart = jnp.minimum(s * trows, n - trows)

        for hh in range(2):
            pltpu.sync_copy(table_h.at[hh, pl.ds(tstart, trows)],
                            tbl.at[pl.ds(tstart, trows)])
            pltpu.sync_copy(z_h, acc.at[pl.ds(s * rpt, rpt)])
            plsc.subcore_barrier()
            for p in range(lag):
                pltpu.async_copy(tbl.at[src_v.at[p]], rows[p], gsem[p])

            @pl.loop(0, ch, step=nb)
            def _(j):
                for b in range(nb):
                    cidx = j + b
                    pltpu.make_async_copy(tbl.at[src_v.at[cidx]], rows[b],
                                          gsem[b]).wait()
                    pltpu.async_copy(rows[b], acc.at[dst_v.at[cidx]],
                                     ssem[b], add=True)
                    fc = cidx + lag
                    fb = (b + lag) % nb

                    @pl.when(fc < ch)
                    def _():
                        @pl.when(fc >= nb)
                        def _():
                            pltpu.make_async_copy(
                                rows[fb], acc.at[dst_v.at[cidx]],
                                ssem[fb]).wait()
                        pltpu.async_copy(tbl.at[src_v.at[fc]], rows[fb],
                                         gsem[fb])

            for b in range(nb):
                pltpu.make_async_copy(rows[b], acc.at[dst_v.at[0]],
                                      ssem[b]).wait()
            plsc.subcore_barrier()
            pltpu.sync_copy(acc.at[pl.ds(s * rpt, rpt)],
                            out_h.at[c, hh, pl.ds(s * rpt, rpt)])
            if hh == 0:
                plsc.subcore_barrier()

    return k


def _deg_rows(nrows, w, ch):
    mesh = plsc.VectorSubcoreMesh(core_axis_name="c", subcore_axis_name="s",
                                  num_cores=NC, num_subcores=NS)
    rpt = nrows // NS

    @functools.partial(
        pl.kernel,
        out_type=jax.ShapeDtypeStruct((NC, nrows, w), jnp.float32),
        mesh=mesh,
        compiler_params=_SC_PARAMS,
        scratch_types=[
            pltpu.VMEM((ch, LK), jnp.int32),
            pltpu.VMEM((LK, w), jnp.float32),
            pltpu.VMEM_SHARED((nrows, w), jnp.float32),
        ],
    )
    def k(dst_h, ones_h, z_h, out_h, dst_v, ones_v, acc):
        c = lax.axis_index("c")
        s = lax.axis_index("s")
        pltpu.sync_copy(dst_h.at[c, s], dst_v)
        pltpu.sync_copy(ones_h, ones_v)
        pltpu.sync_copy(z_h, acc.at[pl.ds(s * rpt, rpt)])
        plsc.subcore_barrier()

        @pl.loop(0, ch)
        def _(j):
            pltpu.sync_copy(ones_v, acc.at[dst_v.at[j]], add=True)

        plsc.subcore_barrier()
        pltpu.sync_copy(acc.at[pl.ds(s * rpt, rpt)],
                        out_h.at[c, pl.ds(s * rpt, rpt)])

    return k


def _enc_body(x_ref, w_ref, b_ref, h_ref):
    h_ref[...] = jnp.maximum(
        jnp.dot(x_ref[...], w_ref[...], preferred_element_type=jnp.float32)
        + b_ref[...], 0.0)


def _pre0_body(deg_ref, h_ref, w_ref, dis_ref, hw_ref, *, n):
    deg = deg_ref[0, :n, 0:1] + deg_ref[1, :n, 0:1] + 1.0
    dis = lax.rsqrt(deg)
    dis_ref[...] = dis
    hw_ref[...] = jnp.dot(h_ref[...], w_ref[...],
                          preferred_element_type=jnp.float32) * dis


def _mid_body(sp_ref, hwp_ref, dis_ref, hres_ref, bc_ref, g_ref, be_ref,
              wn_ref, h_ref, hwn_ref, *, n, residual):
    dis = dis_ref[...]
    ssum = jnp.concatenate(
        [sp_ref[0, 0, :n] + sp_ref[1, 0, :n],
         sp_ref[0, 1, :n] + sp_ref[1, 1, :n]], axis=1) + hwp_ref[...]
    agg = ssum * dis + bc_ref[...]
    h = jnp.maximum(g_ref[...] * (agg * _BN_C) + be_ref[...], 0.0)
    if residual:
        h = h + hres_ref[...]
    h_ref[...] = h
    hwn_ref[...] = jnp.dot(h, wn_ref[...],
                           preferred_element_type=jnp.float32) * dis


def _head_body(sp_ref, hwp_ref, dis_ref, hres_ref, bc_ref, g_ref, be_ref,
               wa1_ref, ba1_ref, wa2_ref, ba2_ref, wk1_ref, bk1_ref,
               wk2_ref, bk2_ref, out_ref, *, n):
    dis = dis_ref[...]
    ssum = jnp.concatenate(
        [sp_ref[0, 0, :n] + sp_ref[1, 0, :n],
         sp_ref[0, 1, :n] + sp_ref[1, 1, :n]], axis=1) + hwp_ref[...]
    agg = ssum * dis + bc_ref[...]
    h = jnp.maximum(g_ref[...] * (agg * _BN_C) + be_ref[...], 0.0)
    h = h + hres_ref[...]
    a = jnp.dot(jnp.tanh(jnp.dot(h, wa1_ref[...],
                                 preferred_element_type=jnp.float32)
                         + ba1_ref[...]),
                wa2_ref[...], preferred_element_type=jnp.float32) + ba2_ref[...]
    a = a - jnp.max(a, axis=0, keepdims=True)
    ea = jnp.exp(a)
    a = ea / jnp.sum(ea, axis=0, keepdims=True)
    h = h * a
    out_ref[...] = jnp.dot(
        jnp.maximum(jnp.dot(h, wk1_ref[...],
                            preferred_element_type=jnp.float32)
                    + bk1_ref[...], 0.0),
        wk2_ref[...], preferred_element_type=jnp.float32) + bk2_ref[...]


def _tc(body, out_shape, **kw):
    return pl.pallas_call(functools.partial(body, **kw), out_shape=out_shape)


def kernel(x, edge_index, W_ne, b_ne, Wc0, bc0, g0, be0, Wc1, bc1, g1, be1,
           Wc2, bc2, g2, be2, Wa1, ba1, Wa2, ba2, Wk1, bk1, Wk2, bk2):
    n = x.shape[0]
    h_dim = W_ne.shape[1]
    e = edge_index.shape[1]

    ch = 4 * math.ceil(e / (NC * NS * LK * 4))
    epad = NC * NS * ch * LK
    rpt = 8 * math.ceil((n + 8) / (NS * 8))
    nrows = NS * rpt

    src = edge_index[0]
    dst = edge_index[1]
    src_r = jnp.pad(src, (0, epad - e)).reshape(NC, NS, ch, LK)
    dst_r = jnp.pad(dst, (0, epad - e), constant_values=n).reshape(
        NC, NS, ch, LK)

    zrows = jnp.zeros((rpt, h_dim // 2), jnp.float32)
    zdeg = jnp.zeros((rpt, 16), jnp.float32)
    ones16 = jnp.ones((LK, 16), jnp.float32)

    seg = _seg_rows(nrows, h_dim, ch, n)
    deg_parts = _deg_rows(nrows, 16, ch)(dst_r, ones16, zdeg)

    h0 = _tc(_enc_body, jax.ShapeDtypeStruct((n, h_dim), jnp.float32))(
        x, W_ne, b_ne)

    dis, hw0 = _tc(_pre0_body,
                   (jax.ShapeDtypeStruct((n, 1), jnp.float32),
                    jax.ShapeDtypeStruct((n, h_dim), jnp.float32)),
                   n=n)(deg_parts, h0, Wc0)

    hf = h_dim // 2

    def _split(t):
        return jnp.stack([t[:, :hf], t[:, hf:]])

    s0 = seg(_split(hw0), src_r, dst_r, zrows)
    h1, hw1 = _tc(_mid_body,
                  (jax.ShapeDtypeStruct((n, h_dim), jnp.float32),
                   jax.ShapeDtypeStruct((n, h_dim), jnp.float32)),
                  n=n, residual=False)(s0, hw0, dis, h0, bc0, g0, be0, Wc1)

    s1 = seg(_split(hw1), src_r, dst_r, zrows)
    h2, hw2 = _tc(_mid_body,
                  (jax.ShapeDtypeStruct((n, h_dim), jnp.float32),
                   jax.ShapeDtypeStruct((n, h_dim), jnp.float32)),
                  n=n, residual=True)(s1, hw1, dis, h1, bc1, g1, be1, Wc2)

    s2 = seg(_split(hw2), src_r, dst_r, zrows)
    out = _tc(_head_body, jax.ShapeDtypeStruct((n, 2), jnp.float32), n=n)(
        s2, hw2, dis, h2, bc2, g2, be2,
        Wa1, ba1, Wa2, ba2, Wk1, bk1, Wk2, bk2)
    return out


# speedup vs baseline: 2.1649x; 1.2315x over previous
import functools
import math

import jax
import jax.numpy as jnp
from jax import lax
from jax.experimental import pallas as pl
from jax.experimental.pallas import tpu as pltpu
from jax.experimental.pallas import tpu_sc as plsc

NC = 2
NS = 16
LK = 125

_BN_C = (1.0 + 1e-5) ** -0.5
_SC_PARAMS = pltpu.CompilerParams(use_tc_tiling_on_sc=False)


def _seg_rows(nrows, w, ch, n):
    mesh = plsc.VectorSubcoreMesh(core_axis_name="c", subcore_axis_name="s",
                                  num_cores=NC, num_subcores=NS)
    rpt = nrows // NS

    nb = 4
    lag = 2

    @functools.partial(
        pl.kernel,
        out_type=jax.ShapeDtypeStruct((NC, nrows, w), jnp.float32),
        mesh=mesh,
        compiler_params=_SC_PARAMS,
        scratch_types=[
            pltpu.VMEM((ch, LK), jnp.int32),
            pltpu.VMEM((ch, LK), jnp.int32),
            [pltpu.VMEM((LK, w), jnp.float32)] * nb,
            pltpu.VMEM_SHARED((nrows, w), jnp.float32),
            pltpu.VMEM_SHARED((n, w), jnp.float32),
            [pltpu.SemaphoreType.DMA] * nb,
            [pltpu.SemaphoreType.DMA] * nb,
        ],
    )
    def k(table_h, src_h, dst_h, z_h, out_h, src_v, dst_v, rows, acc, tbl,
          gsem, ssem):
        c = lax.axis_index("c")
        s = lax.axis_index("s")
        pltpu.sync_copy(src_h.at[s], src_v)
        pltpu.sync_copy(dst_h.at[s], dst_v)
        trows = 8 * math.ceil(n / (NS * 8))
        tstart = jnp.minimum(s * trows, n - trows)
        pltpu.sync_copy(table_h.at[c, pl.ds(tstart, trows)],
                        tbl.at[pl.ds(tstart, trows)])
        pltpu.sync_copy(z_h, acc.at[pl.ds(s * rpt, rpt)])
        plsc.subcore_barrier()
        for p in range(lag):
            pltpu.async_copy(tbl.at[src_v.at[p]], rows[p], gsem[p])

        @pl.loop(0, ch, step=nb)
        def _(j):
            for b in range(nb):
                cidx = j + b
                pltpu.make_async_copy(tbl.at[src_v.at[cidx]], rows[b],
                                      gsem[b]).wait()
                pltpu.async_copy(rows[b], acc.at[dst_v.at[cidx]],
                                 ssem[b], add=True)
                fc = cidx + lag
                fb = (b + lag) % nb

                @pl.when(fc < ch)
                def _():
                    @pl.when(fc >= nb)
                    def _():
                        pltpu.make_async_copy(
                            rows[fb], acc.at[dst_v.at[cidx]],
                            ssem[fb]).wait()
                    pltpu.async_copy(tbl.at[src_v.at[fc]], rows[fb],
                                     gsem[fb])

        for b in range(nb):
            pltpu.make_async_copy(rows[b], acc.at[dst_v.at[0]],
                                  ssem[b]).wait()
        plsc.subcore_barrier()
        pltpu.sync_copy(acc.at[pl.ds(s * rpt, rpt)],
                        out_h.at[c, pl.ds(s * rpt, rpt)])

    return k


def _deg_rows(nrows, w, ch):
    mesh = plsc.VectorSubcoreMesh(core_axis_name="c", subcore_axis_name="s",
                                  num_cores=NC, num_subcores=NS)
    rpt = nrows // NS
    chc = ch // NC

    @functools.partial(
        pl.kernel,
        out_type=jax.ShapeDtypeStruct((NC, nrows, w), jnp.float32),
        mesh=mesh,
        compiler_params=_SC_PARAMS,
        scratch_types=[
            pltpu.VMEM((ch, LK), jnp.int32),
            pltpu.VMEM((LK, w), jnp.float32),
            pltpu.VMEM_SHARED((nrows, w), jnp.float32),
        ],
    )
    def k(dst_h, ones_h, z_h, out_h, dst_v, ones_v, acc):
        c = lax.axis_index("c")
        s = lax.axis_index("s")
        pltpu.sync_copy(dst_h.at[s], dst_v)
        pltpu.sync_copy(ones_h, ones_v)
        pltpu.sync_copy(z_h, acc.at[pl.ds(s * rpt, rpt)])
        plsc.subcore_barrier()
        base = c * chc

        @pl.loop(0, chc)
        def _(j):
            pltpu.sync_copy(ones_v, acc.at[dst_v.at[base + j]], add=True)

        plsc.subcore_barrier()
        pltpu.sync_copy(acc.at[pl.ds(s * rpt, rpt)],
                        out_h.at[c, pl.ds(s * rpt, rpt)])

    return k


def _enc_body(x_ref, w_ref, b_ref, h_ref):
    h_ref[...] = jnp.maximum(
        jnp.dot(x_ref[...], w_ref[...], preferred_element_type=jnp.float32)
        + b_ref[...], 0.0)


def _pre0_body(deg_ref, h_ref, w_ref, dis_ref, hw_ref, *, n, hf):
    deg = deg_ref[0, :n, 0:1] + deg_ref[1, :n, 0:1] + 1.0
    dis = lax.rsqrt(deg)
    dis_ref[...] = dis
    hw = jnp.dot(h_ref[...], w_ref[...],
                 preferred_element_type=jnp.float32) * dis
    hw_ref[0] = hw[:, :hf]
    hw_ref[1] = hw[:, hf:]


def _mid_body(sp_ref, hwp_ref, dis_ref, hres_ref, bc_ref, g_ref, be_ref,
              wn_ref, h_ref, hwn_ref, *, n, hf, residual):
    dis = dis_ref[...]
    hwp = jnp.concatenate([hwp_ref[0], hwp_ref[1]], axis=1)
    ssum = jnp.concatenate([sp_ref[0, :n], sp_ref[1, :n]], axis=1) + hwp
    agg = ssum * dis + bc_ref[...]
    h = jnp.maximum(g_ref[...] * (agg * _BN_C) + be_ref[...], 0.0)
    if residual:
        h = h + hres_ref[...]
    h_ref[...] = h
    hwn = jnp.dot(h, wn_ref[...], preferred_element_type=jnp.float32) * dis
    hwn_ref[0] = hwn[:, :hf]
    hwn_ref[1] = hwn[:, hf:]


def _head_body(sp_ref, hwp_ref, dis_ref, hres_ref, bc_ref, g_ref, be_ref,
               wa1_ref, ba1_ref, wa2_ref, ba2_ref, wk1_ref, bk1_ref,
               wk2_ref, bk2_ref, out_ref, *, n):
    dis = dis_ref[...]
    hwp = jnp.concatenate([hwp_ref[0], hwp_ref[1]], axis=1)
    ssum = jnp.concatenate([sp_ref[0, :n], sp_ref[1, :n]], axis=1) + hwp
    agg = ssum * dis + bc_ref[...]
    h = jnp.maximum(g_ref[...] * (agg * _BN_C) + be_ref[...], 0.0)
    h = h + hres_ref[...]
    a = jnp.dot(jnp.tanh(jnp.dot(h, wa1_ref[...],
                                 preferred_element_type=jnp.float32)
                         + ba1_ref[...]),
                wa2_ref[...], preferred_element_type=jnp.float32) + ba2_ref[...]
    a = a - jnp.max(a, axis=0, keepdims=True)
    ea = jnp.exp(a)
    a = ea / jnp.sum(ea, axis=0, keepdims=True)
    h = h * a
    out_ref[...] = jnp.dot(
        jnp.maximum(jnp.dot(h, wk1_ref[...],
                            preferred_element_type=jnp.float32)
                    + bk1_ref[...], 0.0),
        wk2_ref[...], preferred_element_type=jnp.float32) + bk2_ref[...]


def _tc(body, out_shape, **kw):
    return pl.pallas_call(functools.partial(body, **kw), out_shape=out_shape)


def kernel(x, edge_index, W_ne, b_ne, Wc0, bc0, g0, be0, Wc1, bc1, g1, be1,
           Wc2, bc2, g2, be2, Wa1, ba1, Wa2, ba2, Wk1, bk1, Wk2, bk2):
    n = x.shape[0]
    h_dim = W_ne.shape[1]
    hf = h_dim // 2
    e = edge_index.shape[1]

    ch = 4 * math.ceil(e / (NS * LK * 4))
    epad = NS * ch * LK
    rpt = 8 * math.ceil((n + 8) / (NS * 8))
    nrows = NS * rpt

    src_r = jnp.pad(edge_index[0], (0, epad - e)).reshape(NS, ch, LK)
    dst_r = jnp.pad(edge_index[1], (0, epad - e),
                    constant_values=n).reshape(NS, ch, LK)

    zrows = jnp.zeros((rpt, hf), jnp.float32)
    zdeg = jnp.zeros((rpt, 16), jnp.float32)
    ones16 = jnp.ones((LK, 16), jnp.float32)

    seg = _seg_rows(nrows, hf, ch, n)
    deg_parts = _deg_rows(nrows, 16, ch)(dst_r, ones16, zdeg)

    h0 = _tc(_enc_body, jax.ShapeDtypeStruct((n, h_dim), jnp.float32))(
        x, W_ne, b_ne)

    dis, hw0 = _tc(_pre0_body,
                   (jax.ShapeDtypeStruct((n, 1), jnp.float32),
                    jax.ShapeDtypeStruct((2, n, hf), jnp.float32)),
                   n=n, hf=hf)(deg_parts, h0, Wc0)

    s0 = seg(hw0, src_r, dst_r, zrows)
    h1, hw1 = _tc(_mid_body,
                  (jax.ShapeDtypeStruct((n, h_dim), jnp.float32),
                   jax.ShapeDtypeStruct((2, n, hf), jnp.float32)),
                  n=n, hf=hf, residual=False)(
        s0, hw0, dis, h0, bc0, g0, be0, Wc1)

    s1 = seg(hw1, src_r, dst_r, zrows)
    h2, hw2 = _tc(_mid_body,
                  (jax.ShapeDtypeStruct((n, h_dim), jnp.float32),
                   jax.ShapeDtypeStruct((2, n, hf), jnp.float32)),
                  n=n, hf=hf, residual=True)(
        s1, hw1, dis, h1, bc1, g1, be1, Wc2)

    s2 = seg(hw2, src_r, dst_r, zrows)
    out = _tc(_head_body, jax.ShapeDtypeStruct((n, 2), jnp.float32), n=n)(
        s2, hw2, dis, h2, bc2, g2, be2,
        Wa1, ba1, Wa2, ba2, Wk1, bk1, Wk2, bk2)
    return out


# speedup vs baseline: 2.1821x; 1.0079x over previous
import functools
import math

import jax
import jax.numpy as jnp
from jax import lax
from jax.experimental import pallas as pl
from jax.experimental.pallas import tpu as pltpu
from jax.experimental.pallas import tpu_sc as plsc

NC = 2
NS = 16
LK = 125

_BN_C = (1.0 + 1e-5) ** -0.5
_SC_PARAMS = pltpu.CompilerParams(use_tc_tiling_on_sc=False)


def _seg_rows(nrows, w, ch, n):
    mesh = plsc.VectorSubcoreMesh(core_axis_name="c", subcore_axis_name="s",
                                  num_cores=NC, num_subcores=NS)
    rpt = nrows // NS

    nb = 8
    lag = 4

    @functools.partial(
        pl.kernel,
        out_type=jax.ShapeDtypeStruct((NC, nrows, w), jnp.float32),
        mesh=mesh,
        compiler_params=_SC_PARAMS,
        scratch_types=[
            pltpu.VMEM((ch, LK), jnp.int32),
            pltpu.VMEM((ch, LK), jnp.int32),
            [pltpu.VMEM((LK, w), jnp.float32)] * nb,
            pltpu.VMEM_SHARED((nrows, w), jnp.float32),
            pltpu.VMEM_SHARED((n, w), jnp.float32),
            [pltpu.SemaphoreType.DMA] * nb,
            [pltpu.SemaphoreType.DMA] * nb,
        ],
    )
    def k(table_h, src_h, dst_h, z_h, out_h, src_v, dst_v, rows, acc, tbl,
          gsem, ssem):
        c = lax.axis_index("c")
        s = lax.axis_index("s")
        pltpu.sync_copy(src_h.at[s], src_v)
        pltpu.sync_copy(dst_h.at[s], dst_v)
        trows = 8 * math.ceil(n / (NS * 8))
        tstart = jnp.minimum(s * trows, n - trows)
        pltpu.sync_copy(table_h.at[c, pl.ds(tstart, trows)],
                        tbl.at[pl.ds(tstart, trows)])
        pltpu.sync_copy(z_h, acc.at[pl.ds(s * rpt, rpt)])
        plsc.subcore_barrier()
        for p in range(lag):
            pltpu.async_copy(tbl.at[src_v.at[p]], rows[p], gsem[p])

        @pl.loop(0, ch, step=nb)
        def _(j):
            for b in range(nb):
                cidx = j + b
                pltpu.make_async_copy(tbl.at[src_v.at[cidx]], rows[b],
                                      gsem[b]).wait()
                pltpu.async_copy(rows[b], acc.at[dst_v.at[cidx]],
                                 ssem[b], add=True)
                fc = cidx + lag
                fb = (b + lag) % nb

                @pl.when(fc < ch)
                def _():
                    @pl.when(fc >= nb)
                    def _():
                        pltpu.make_async_copy(
                            rows[fb], acc.at[dst_v.at[cidx]],
                            ssem[fb]).wait()
                    pltpu.async_copy(tbl.at[src_v.at[fc]], rows[fb],
                                     gsem[fb])

        for b in range(nb):
            pltpu.make_async_copy(rows[b], acc.at[dst_v.at[0]],
                                  ssem[b]).wait()
        plsc.subcore_barrier()
        pltpu.sync_copy(acc.at[pl.ds(s * rpt, rpt)],
                        out_h.at[c, pl.ds(s * rpt, rpt)])

    return k


def _deg_rows(nrows, w, ch):
    mesh = plsc.VectorSubcoreMesh(core_axis_name="c", subcore_axis_name="s",
                                  num_cores=NC, num_subcores=NS)
    rpt = nrows // NS
    chc = ch // NC

    @functools.partial(
        pl.kernel,
        out_type=jax.ShapeDtypeStruct((NC, nrows, w), jnp.float32),
        mesh=mesh,
        compiler_params=_SC_PARAMS,
        scratch_types=[
            pltpu.VMEM((ch, LK), jnp.int32),
            pltpu.VMEM((LK, w), jnp.float32),
            pltpu.VMEM_SHARED((nrows, w), jnp.float32),
        ],
    )
    def k(dst_h, ones_h, z_h, out_h, dst_v, ones_v, acc):
        c = lax.axis_index("c")
        s = lax.axis_index("s")
        pltpu.sync_copy(dst_h.at[s], dst_v)
        pltpu.sync_copy(ones_h, ones_v)
        pltpu.sync_copy(z_h, acc.at[pl.ds(s * rpt, rpt)])
        plsc.subcore_barrier()
        base = c * chc

        @pl.loop(0, chc)
        def _(j):
            pltpu.sync_copy(ones_v, acc.at[dst_v.at[base + j]], add=True)

        plsc.subcore_barrier()
        pltpu.sync_copy(acc.at[pl.ds(s * rpt, rpt)],
                        out_h.at[c, pl.ds(s * rpt, rpt)])

    return k


def _enc_body(x_ref, w_ref, b_ref, h_ref):
    h_ref[...] = jnp.maximum(
        jnp.dot(x_ref[...], w_ref[...], preferred_element_type=jnp.float32)
        + b_ref[...], 0.0)


def _pre0_body(deg_ref, h_ref, w_ref, dis_ref, hw_ref, *, n, hf):
    deg = deg_ref[0, :n, 0:1] + deg_ref[1, :n, 0:1] + 1.0
    dis = lax.rsqrt(deg)
    dis_ref[...] = dis
    hw = jnp.dot(h_ref[...], w_ref[...],
                 preferred_element_type=jnp.float32) * dis
    hw_ref[0] = hw[:, :hf]
    hw_ref[1] = hw[:, hf:]


def _mid_body(sp_ref, hwp_ref, dis_ref, hres_ref, bc_ref, g_ref, be_ref,
              wn_ref, h_ref, hwn_ref, *, n, hf, residual):
    dis = dis_ref[...]
    hwp = jnp.concatenate([hwp_ref[0], hwp_ref[1]], axis=1)
    ssum = jnp.concatenate([sp_ref[0, :n], sp_ref[1, :n]], axis=1) + hwp
    agg = ssum * dis + bc_ref[...]
    h = jnp.maximum(g_ref[...] * (agg * _BN_C) + be_ref[...], 0.0)
    if residual:
        h = h + hres_ref[...]
    h_ref[...] = h
    hwn = jnp.dot(h, wn_ref[...], preferred_element_type=jnp.float32) * dis
    hwn_ref[0] = hwn[:, :hf]
    hwn_ref[1] = hwn[:, hf:]


def _head_body(sp_ref, hwp_ref, dis_ref, hres_ref, bc_ref, g_ref, be_ref,
               wa1_ref, ba1_ref, wa2_ref, ba2_ref, wk1_ref, bk1_ref,
               wk2_ref, bk2_ref, out_ref, *, n):
    dis = dis_ref[...]
    hwp = jnp.concatenate([hwp_ref[0], hwp_ref[1]], axis=1)
    ssum = jnp.concatenate([sp_ref[0, :n], sp_ref[1, :n]], axis=1) + hwp
    agg = ssum * dis + bc_ref[...]
    h = jnp.maximum(g_ref[...] * (agg * _BN_C) + be_ref[...], 0.0)
    h = h + hres_ref[...]
    a = jnp.dot(jnp.tanh(jnp.dot(h, wa1_ref[...],
                                 preferred_element_type=jnp.float32)
                         + ba1_ref[...]),
                wa2_ref[...], preferred_element_type=jnp.float32) + ba2_ref[...]
    a = a - jnp.max(a, axis=0, keepdims=True)
    ea = jnp.exp(a)
    a = ea / jnp.sum(ea, axis=0, keepdims=True)
    h = h * a
    out_ref[...] = jnp.dot(
        jnp.maximum(jnp.dot(h, wk1_ref[...],
                            preferred_element_type=jnp.float32)
                    + bk1_ref[...], 0.0),
        wk2_ref[...], preferred_element_type=jnp.float32) + bk2_ref[...]


def _tc(body, out_shape, **kw):
    return pl.pallas_call(functools.partial(body, **kw), out_shape=out_shape)


def kernel(x, edge_index, W_ne, b_ne, Wc0, bc0, g0, be0, Wc1, bc1, g1, be1,
           Wc2, bc2, g2, be2, Wa1, ba1, Wa2, ba2, Wk1, bk1, Wk2, bk2):
    n = x.shape[0]
    h_dim = W_ne.shape[1]
    hf = h_dim // 2
    e = edge_index.shape[1]

    ch = 8 * math.ceil(e / (NS * LK * 8))
    epad = NS * ch * LK
    rpt = 8 * math.ceil((n + 8) / (NS * 8))
    nrows = NS * rpt

    src_r = jnp.pad(edge_index[0], (0, epad - e)).reshape(NS, ch, LK)
    dst_r = jnp.pad(edge_index[1], (0, epad - e),
                    constant_values=n).reshape(NS, ch, LK)

    zrows = jnp.zeros((rpt, hf), jnp.float32)
    zdeg = jnp.zeros((rpt, 16), jnp.float32)
    ones16 = jnp.ones((LK, 16), jnp.float32)

    seg = _seg_rows(nrows, hf, ch, n)
    deg_parts = _deg_rows(nrows, 16, ch)(dst_r, ones16, zdeg)

    h0 = _tc(_enc_body, jax.ShapeDtypeStruct((n, h_dim), jnp.float32))(
        x, W_ne, b_ne)

    dis, hw0 = _tc(_pre0_body,
                   (jax.ShapeDtypeStruct((n, 1), jnp.float32),
                    jax.ShapeDtypeStruct((2, n, hf), jnp.float32)),
                   n=n, hf=hf)(deg_parts, h0, Wc0)

    s0 = seg(hw0, src_r, dst_r, zrows)
    h1, hw1 = _tc(_mid_body,
                  (jax.ShapeDtypeStruct((n, h_dim), jnp.float32),
                   jax.ShapeDtypeStruct((2, n, hf), jnp.float32)),
                  n=n, hf=hf, residual=False)(
        s0, hw0, dis, h0, bc0, g0, be0, Wc1)

    s1 = seg(hw1, src_r, dst_r, zrows)
    h2, hw2 = _tc(_mid_body,
                  (jax.ShapeDtypeStruct((n, h_dim), jnp.float32),
                   jax.ShapeDtypeStruct((2, n, hf), jnp.float32)),
                  n=n, hf=hf, residual=True)(
        s1, hw1, dis, h1, bc1, g1, be1, Wc2)

    s2 = seg(hw2, src_r, dst_r, zrows)
    out = _tc(_head_body, jax.ShapeDtypeStruct((n, 2), jnp.float32), n=n)(
        s2, hw2, dis, h2, bc2, g2, be2,
        Wa1, ba1, Wa2, ba2, Wk1, bk1, Wk2, bk2)
    return out
